# Initial kernel scaffold; baseline (speedup 1.0000x reference)
import jax
import jax.numpy as jnp
from jax.experimental import pallas as pl


def kernel(x, coords_input, coords_output, adjc, W_lift, W_coord_in, W_coord_out, W_es1, W_en1, W_es2, W_en2, W_q, W_post, W_ds1, W_dn1, W_ds2, W_dn2, W_out, eps):
    raise NotImplementedError("write your pallas kernel here")


# speedup vs baseline: 1.6335x; 1.6335x over previous
import functools

import jax
import jax.numpy as jnp
from jax import lax
from jax.experimental import pallas as pl
from jax.experimental.pallas import tpu as pltpu
from jax.experimental.pallas import tpu_sc as plsc

N = 10000
K = 32
D = 128
LAT = 64

NC = 2
NS = 16
NW = NC * NS
NODES_PW = 320
NPAD = NW * NODES_PW
CHUNK = 4
RPC = CHUNK * K
NCH = NODES_PW // CHUNK
INV_K = 1.0 / K

@functools.cache
def _build_gather_mean():
    mesh = plsc.VectorSubcoreMesh(core_axis_name="c", subcore_axis_name="s")
    return functools.partial(
        pl.kernel,
        mesh=mesh,
        out_type=jax.ShapeDtypeStruct((NPAD, D), jnp.float32),
        scratch_types=[
            pltpu.VMEM((NODES_PW * K,), jnp.int32),
            pltpu.VMEM((2, RPC, D), jnp.float32),
            pltpu.VMEM((2, CHUNK, D), jnp.float32),
            pltpu.SemaphoreType.DMA,
            pltpu.SemaphoreType.DMA,
            pltpu.SemaphoreType.DMA,
            pltpu.SemaphoreType.DMA,
        ],
    )(_gather_mean_body)


def _gather_mean(p, idx):
    return _build_gather_mean()(p, idx)


def _gather_mean_body(p_hbm, idx_hbm, out_hbm, idx_v, rows_v, stage_v, gsem0, gsem1, osem0, osem1):
    wid = lax.axis_index("s") * NC + lax.axis_index("c")
    node_base = wid * NODES_PW

    pltpu.sync_copy(idx_hbm.at[pl.ds(node_base * K, NODES_PW * K)], idx_v)

    gsems = (gsem0, gsem1)
    osems = (osem0, osem1)

    def start_gather(c, b):
        pltpu.async_copy(
            p_hbm.at[idx_v.at[pl.ds(c * RPC, RPC)]], rows_v.at[b], gsems[b])

    def wait_gather(b):
        pltpu.make_async_copy(
            p_hbm.at[pl.ds(0, RPC)], rows_v.at[b], gsems[b]).wait()

    def start_out(c, b):
        pltpu.async_copy(
            stage_v.at[b], out_hbm.at[pl.ds(node_base + c * CHUNK, CHUNK)], osems[b])

    def wait_out(b):
        pltpu.make_async_copy(
            stage_v.at[b], out_hbm.at[pl.ds(0, CHUNK)], osems[b]).wait()

    def accumulate(b):
        for j in range(CHUNK):
            base = j * K

            def kstep(k, accs):
                out = []
                for cc in range(D // 16):
                    a = accs[cc]
                    for u in range(4):
                        a = a + rows_v[b, base + 4 * k + u, pl.ds(cc * 16, 16)]
                    out.append(a)
                return tuple(out)

            zeros = tuple(jnp.zeros((16,), jnp.float32) for _ in range(D // 16))
            accs = lax.fori_loop(0, K // 4, kstep, zeros)
            for cc in range(D // 16):
                stage_v[b, j, pl.ds(cc * 16, 16)] = accs[cc] * INV_K

    start_gather(0, 0)
    start_gather(1, 1)

    def body(i, carry):
        for b in range(2):
            c = 2 * i + b
            wait_gather(b)

            @pl.when(i > 0)
            def _():
                wait_out(b)

            accumulate(b)

            @pl.when(c + 2 < NCH)
            def _():
                start_gather(c + 2, b)

            start_out(c, b)
        return carry

    lax.fori_loop(0, NCH // 2, body, 0)
    wait_out(0)
    wait_out(1)


_BLK = 2000
_GRID = N // _BLK


def _row_spec(cols):
    return pl.BlockSpec((_BLK, cols), lambda i: (i, 0))


def _full_spec(r, c):
    return pl.BlockSpec((r, c), lambda i: (0, 0))


def _dot(a, b):
    return jnp.dot(a, b, preferred_element_type=jnp.float32)


def _tc_lift_body(x_ref, ci_ref, wl_ref, wci_ref, wn_ref, h_ref, p_ref):
    h = x_ref[...] * wl_ref[...] + _dot(ci_ref[...], wci_ref[...])
    h_ref[...] = h
    p_ref[...] = _dot(h, wn_ref[...])


def _tc_lift(xc, ci, wl, wci, wn):
    return pl.pallas_call(
        _tc_lift_body,
        grid=(_GRID,),
        in_specs=[_row_spec(1), _row_spec(2), _full_spec(1, D), _full_spec(2, D),
                  _full_spec(D, D)],
        out_specs=[_row_spec(D), _row_spec(D)],
        out_shape=[jax.ShapeDtypeStruct((N, D), jnp.float32)] * 2,
    )(xc, ci, wl, wci, wn)


def _tc_mp_body(h_ref, m_ref, ws_ref, wn_ref, h_out_ref, p_out_ref):
    hn = jax.nn.gelu(_dot(h_ref[...], ws_ref[...]) + m_ref[...])
    h_out_ref[...] = hn
    p_out_ref[...] = _dot(hn, wn_ref[...])


def _tc_mp(h, m, ws, wn):
    return pl.pallas_call(
        _tc_mp_body,
        grid=(_GRID,),
        in_specs=[_row_spec(D), _row_spec(D), _full_spec(D, D), _full_spec(D, D)],
        out_specs=[_row_spec(D), _row_spec(D)],
        out_shape=[jax.ShapeDtypeStruct((N, D), jnp.float32)] * 2,
    )(h, m, ws, wn)


def _tc_mid_body(h_ref, m_ref, ws_ref, wq_ref, wpost_ref, co_ref, wco_ref,
                 eps_ref, wn_ref, mom_ref, g_ref, p_ref):
    h2 = jax.nn.gelu(_dot(h_ref[...], ws_ref[...]) + m_ref[...])
    mom = _dot(h2, wq_ref[...])
    mu = mom[:, :LAT]
    logvar = jnp.clip(mom[:, LAT:], -30.0, 20.0)
    z = mu + jnp.exp(0.5 * logvar) * eps_ref[...]
    g = _dot(z, wpost_ref[...]) + _dot(co_ref[...], wco_ref[...])
    mom_ref[...] = jnp.concatenate([mu, logvar], axis=1)
    g_ref[...] = g
    p_ref[...] = _dot(g, wn_ref[...])


def _tc_mid(h, m, ws, wq, wpost, co, wco, eps, wn):
    return pl.pallas_call(
        _tc_mid_body,
        grid=(_GRID,),
        in_specs=[_row_spec(D), _row_spec(D), _full_spec(D, D),
                  _full_spec(D, 2 * LAT), _full_spec(LAT, D), _row_spec(2),
                  _full_spec(2, D), _row_spec(LAT), _full_spec(D, D)],
        out_specs=[_row_spec(2 * LAT), _row_spec(D), _row_spec(D)],
        out_shape=[jax.ShapeDtypeStruct((N, 2 * LAT), jnp.float32),
                   jax.ShapeDtypeStruct((N, D), jnp.float32),
                   jax.ShapeDtypeStruct((N, D), jnp.float32)],
    )(h, m, ws, wq, wpost, co, wco, eps, wn)


def _tc_out_body(g_ref, m_ref, ws_ref, wout_ref, dec_ref):
    g2 = jax.nn.gelu(_dot(g_ref[...], ws_ref[...]) + m_ref[...])
    dec_ref[...] = jnp.sum(g2 * wout_ref[...], axis=1, keepdims=True)


def _tc_out(g, m, ws, wout_row):
    return pl.pallas_call(
        _tc_out_body,
        grid=(_GRID,),
        in_specs=[_row_spec(D), _row_spec(D), _full_spec(D, D), _full_spec(1, D)],
        out_specs=[_row_spec(1)],
        out_shape=[jax.ShapeDtypeStruct((N, 1), jnp.float32)],
    )(g, m, ws, wout_row)[0]


def kernel(x, coords_input, coords_output, adjc, W_lift, W_coord_in, W_coord_out,
           W_es1, W_en1, W_es2, W_en2, W_q, W_post,
           W_ds1, W_dn1, W_ds2, W_dn2, W_out, eps):
    b = x.shape[0]
    xc = x.reshape(N, 1)
    idx = jnp.pad(adjc.reshape(-1), (0, NPAD * K - N * K))

    h0, p0 = _tc_lift(xc, coords_input, W_lift, W_coord_in, W_en1)
    m1 = _gather_mean(p0, idx)[:N]
    h1, p1 = _tc_mp(h0, m1, W_es1, W_en2)
    m2 = _gather_mean(p1, idx)[:N]
    mom, g0, p2 = _tc_mid(h1, m2, W_es2, W_q, W_post, coords_output,
                          W_coord_out, eps, W_dn1)
    m3 = _gather_mean(p2, idx)[:N]
    g1, p3 = _tc_mp(g0, m3, W_ds1, W_dn2)
    m4 = _gather_mean(p3, idx)[:N]
    dec = _tc_out(g1, m4, W_ds2, W_out.reshape(1, D))

    return dec.reshape(b, N, -1), mom[:, :LAT], mom[:, LAT:]
